# baseline (speedup 1.0000x reference)
import math

import jax
import jax.numpy as jnp
from jax import lax
from jax.experimental import pallas as pl
from jax.experimental.pallas import tpu as pltpu
from jax.experimental.pallas import tpu_sc as plsc

VOCAB = 100000
EMBED_DIM = 128
BATCH = 1024
SEQ = 200

_NC = 2
_NS = 16
_NW = _NC * _NS

_TOTAL = BATCH * SEQ
_CHUNK = 80
_PER_W = _TOTAL // _NW
_NCHUNK = _PER_W // _CHUNK

_K = 2
_NBANK = 4
_NG = _NCHUNK // _K
_NR = _NG // _NBANK


def _sc_gather_body(table_hbm, idx_hbm, out_hbm, idx_v,
                    bank0, bank1, bank2, bank3,
                    sg0, sg1, sg2, sg3, sp0, sp1, sp2, sp3):
    banks = (bank0, bank1, bank2, bank3)
    semg = (sg0, sg1, sg2, sg3)
    semp = (sp0, sp1, sp2, sp3)

    wid = lax.axis_index("s") * _NC + lax.axis_index("c")
    base = wid * _PER_W
    pltpu.sync_copy(idx_hbm.at[pl.ds(base, _PER_W)], idx_v)

    def _gather_copy(c, x, k):
        return pltpu.make_async_copy(
            table_hbm.at[idx_v.at[pl.ds(c * _CHUNK, _CHUNK)]],
            banks[x].at[k],
            semg[x],
        )

    def _put_copy(c, x, k):
        return pltpu.make_async_copy(
            banks[x].at[k],
            out_hbm.at[pl.ds(base + c * _CHUNK, _CHUNK)],
            semp[x],
        )

    def group_gathers(j, x):
        for k in range(_K):
            _gather_copy(j * _K + k, x, k).start()

    def group_drain_gathers(j, x):
        for k in range(_K):
            _gather_copy(j * _K + k, x, k).wait()

    def group_puts(j, x):
        for k in range(_K):
            _put_copy(j * _K + k, x, k).start()

    def group_drain_puts(j, x):
        for k in range(_K):
            _put_copy(j * _K + k, x, k).wait()

    group_gathers(0, 0)
    for x in range(1, _NBANK):
        group_gathers(x, x)
        group_drain_gathers(x - 1, x - 1)
        group_puts(x - 1, x - 1)

    def steady(r, _):
        jbase = r * _NBANK
        for x in range(_NBANK):
            j = jbase + x
            xp = (x - 1) % _NBANK
            group_drain_puts(j - _NBANK, x)
            group_gathers(j, x)
            group_drain_gathers(j - 1, xp)
            group_puts(j - 1, xp)
        return 0

    lax.fori_loop(1, _NR, steady, 0)

    xl = (_NG - 1) % _NBANK
    group_drain_gathers(_NG - 1, xl)
    group_puts(_NG - 1, xl)
    for j in range(_NG - _NBANK, _NG):
        group_drain_puts(j, j % _NBANK)


@jax.jit
def _sc_gather(table, idx_flat):
    mesh = plsc.VectorSubcoreMesh(core_axis_name="c", subcore_axis_name="s")
    bank = pltpu.VMEM((_K, _CHUNK, EMBED_DIM), jnp.float32)
    return pl.kernel(
        _sc_gather_body,
        out_type=jax.ShapeDtypeStruct((_TOTAL, EMBED_DIM), jnp.float32),
        mesh=mesh,
        scratch_types=(
            [pltpu.VMEM((_PER_W,), jnp.int32)]
            + [bank] * _NBANK
            + [pltpu.SemaphoreType.DMA] * (2 * _NBANK)
        ),
    )(table, idx_flat)


def _pe_body(out_ref):
    shape = (SEQ, EMBED_DIM)
    pos = lax.broadcasted_iota(jnp.int32, shape, 0).astype(jnp.float32)
    i = lax.broadcasted_iota(jnp.int32, shape, 1)
    two_floor = (2 * (i // 2)).astype(jnp.float32)
    inv_freq = jnp.exp(two_floor * (-math.log(10000.0) / float(EMBED_DIM)))
    angle = pos * inv_freq
    odd = (i % 2) == 1
    out_ref[...] = jnp.where(odd, jnp.cos(angle), jnp.sin(angle))


@jax.jit
def _pos_encoding():
    return pl.pallas_call(
        _pe_body,
        out_shape=jax.ShapeDtypeStruct((SEQ, EMBED_DIM), jnp.float32),
    )()


def kernel(inputs, table):
    idx_flat = inputs.reshape(_TOTAL)
    embed = _sc_gather(table, idx_flat)
    pe = _pos_encoding()
    return embed.reshape(BATCH, SEQ, EMBED_DIM), pe
